# baseline (speedup 1.0000x reference)
import functools

import jax
import jax.numpy as jnp
from jax import lax
from jax.experimental import pallas as pl
from jax.experimental.pallas import tpu as pltpu
from jax.experimental.pallas import tpu_sc as plsc

N = 10000
E = 320000
D_IN = 256
HID = 16
D_EDGE = 16

NC = 2
NS = 16
NW = NC * NS
CHUNK = 128
CPT = 80
DEPTH = 2
NGRP = CPT // DEPTH
ADEPTH = 8
ANGRP = CPT // ADEPTH
E_PAD = NW * CPT * CHUNK
NCHK = E_PAD // CHUNK
NCHK_R = E // CHUNK
N_PAD = 10112
RPS = N_PAD // NS

_MESH = dict(core_axis_name="c", subcore_axis_name="s")


def _wid():
    return lax.axis_index("s") * NC + lax.axis_index("c")


def _zero_spmem(zbuf_v, acc_sh):
    sid = lax.axis_index("s")

    def fill(i, _):
        zbuf_v[i] = jnp.zeros((16,), jnp.float32)
        return 0

    lax.fori_loop(0, RPS, fill, 0)
    pltpu.sync_copy(zbuf_v, acc_sh.at[pl.ds(sid * RPS, RPS)])


def _write_out(acc_sh, out_hbm):
    cid = lax.axis_index("c")
    sid = lax.axis_index("s")
    pltpu.sync_copy(acc_sh.at[pl.ds(sid * RPS, RPS)],
                    out_hbm.at[cid, pl.ds(sid * RPS, RPS)])


@functools.partial(
    pl.kernel,
    out_type=jax.ShapeDtypeStruct((NC, N_PAD, 16), jnp.float32),
    mesh=plsc.VectorSubcoreMesh(**_MESH),
    compiler_params=pltpu.CompilerParams(use_tc_tiling_on_sc=False),
    scratch_types=[
        pltpu.VMEM((CPT, CHUNK), jnp.int32),
        pltpu.VMEM((CHUNK, 16), jnp.float32),
        pltpu.VMEM((RPS, 16), jnp.float32),
        pltpu.VMEM_SHARED((N_PAD, 16), jnp.float32),
        pltpu.SemaphoreType.DMA,
    ],
)
def _deg_kernel(dst_hbm, out_hbm, dst_v, ones_v, zbuf_v, acc_sh, sem_s):
    wid = _wid()

    def fill(i, _):
        ones_v[i] = jnp.ones((16,), jnp.float32)
        return 0

    lax.fori_loop(0, CHUNK, fill, 0)
    _zero_spmem(zbuf_v, acc_sh)
    plsc.subcore_barrier()
    pltpu.sync_copy(dst_hbm.at[wid], dst_v)

    def body(i, _):
        for c in range(ADEPTH):
            pltpu.async_copy(ones_v, acc_sh.at[dst_v.at[i * ADEPTH + c]],
                             sem_s, add=True)
        for c in range(ADEPTH):
            pltpu.make_async_copy(
                ones_v, acc_sh.at[dst_v.at[i * ADEPTH + c]], sem_s).wait()
        return 0

    lax.fori_loop(0, ANGRP, body, 0)
    plsc.subcore_barrier()
    _write_out(acc_sh, out_hbm)


@functools.partial(
    pl.kernel,
    out_type=jax.ShapeDtypeStruct((NC, N_PAD, 16), jnp.float32),
    mesh=plsc.VectorSubcoreMesh(**_MESH),
    compiler_params=pltpu.CompilerParams(use_tc_tiling_on_sc=False),
    scratch_types=[
        pltpu.VMEM((CPT, CHUNK), jnp.int32),
        pltpu.VMEM((CPT, CHUNK), jnp.int32),
        pltpu.VMEM((ADEPTH, CHUNK, 16), jnp.float32),
        pltpu.VMEM((RPS, 16), jnp.float32),
        pltpu.VMEM_SHARED((N_PAD, 16), jnp.float32),
        pltpu.SemaphoreType.DMA,
        pltpu.SemaphoreType.DMA,
    ],
)
def _agg_kernel(table_hbm, src_hbm, dst_hbm, out_hbm,
                src_v, dst_v, rows_v, zbuf_v, acc_sh, sem_g, sem_s):
    wid = _wid()
    _zero_spmem(zbuf_v, acc_sh)
    plsc.subcore_barrier()
    pltpu.sync_copy(src_hbm.at[wid], src_v)
    pltpu.sync_copy(dst_hbm.at[wid], dst_v)

    for c in range(ADEPTH):
        pltpu.async_copy(table_hbm.at[src_v.at[c]], rows_v.at[c], sem_g)

    def body(i, _):
        for c in range(ADEPTH):
            j = i * ADEPTH + c
            pltpu.make_async_copy(
                table_hbm.at[src_v.at[j]], rows_v.at[c], sem_g).wait()
            pltpu.async_copy(rows_v.at[c], acc_sh.at[dst_v.at[j]],
                             sem_s, add=True)
        for c in range(ADEPTH):
            j = i * ADEPTH + c
            pltpu.make_async_copy(
                rows_v.at[c], acc_sh.at[dst_v.at[j]], sem_s).wait()

            @pl.when(i < ANGRP - 1)
            def _():
                pltpu.async_copy(
                    table_hbm.at[src_v.at[j + ADEPTH]], rows_v.at[c], sem_g)
        return 0

    lax.fori_loop(0, ANGRP, body, 0)
    plsc.subcore_barrier()
    _write_out(acc_sh, out_hbm)


@functools.partial(
    pl.kernel,
    out_type=jax.ShapeDtypeStruct((2, NCHK, 8, CHUNK), jnp.float32),
    mesh=plsc.VectorSubcoreMesh(**_MESH),
    compiler_params=pltpu.CompilerParams(use_tc_tiling_on_sc=False,
                                         needs_layout_passes=False),
    scratch_types=[
        pltpu.VMEM((CPT, CHUNK), jnp.int32),
        pltpu.VMEM((CPT, CHUNK), jnp.int32),
        pltpu.VMEM((DEPTH, CHUNK, 16), jnp.float32),
        pltpu.VMEM((DEPTH, CHUNK, 16), jnp.float32),
        pltpu.VMEM((DEPTH, 16, CHUNK), jnp.float32),
        pltpu.SemaphoreType.DMA,
        pltpu.SemaphoreType.DMA,
        pltpu.SemaphoreType.DMA,
    ],
)
def _gathp_kernel(h2_hbm, src_hbm, dst_hbm, out_hbm,
                  src_v, dst_v, rs_v, rd_v, ods_v, sem_g, sem_g2, sem_w):
    wid = _wid()
    pltpu.sync_copy(src_hbm.at[wid], src_v)
    pltpu.sync_copy(dst_hbm.at[wid], dst_v)
    cbase = wid * CPT
    iota16 = lax.iota(jnp.int32, 16)
    dsels = [(iota16 + k) & 15 for k in range(16)]
    rowis = [g * 16 + iota16 for g in range(8)]

    for c in range(DEPTH):
        pltpu.async_copy(h2_hbm.at[src_v.at[c]], rs_v.at[c], sem_g)
        pltpu.async_copy(h2_hbm.at[dst_v.at[c]], rd_v.at[c], sem_g2)

    def body(i, _):
        for c in range(DEPTH):
            j = i * DEPTH + c

            @pl.when(i > 0)
            def _():
                pltpu.make_async_copy(
                    ods_v.at[c, pl.ds(0, 8)],
                    out_hbm.at[0, cbase + j - DEPTH], sem_w).wait()
                pltpu.make_async_copy(
                    ods_v.at[c, pl.ds(8, 8)],
                    out_hbm.at[1, cbase + j - DEPTH], sem_w).wait()

            pltpu.make_async_copy(
                h2_hbm.at[src_v.at[j]], rs_v.at[c], sem_g).wait()
            pltpu.make_async_copy(
                h2_hbm.at[dst_v.at[j]], rd_v.at[c], sem_g2).wait()

            for g in range(8):
                for k in range(16):
                    a = plsc.load_gather(rs_v.at[c], [rowis[g], dsels[k]])
                    b = plsc.load_gather(rd_v.at[c], [rowis[g], dsels[k]])
                    plsc.store_scatter(ods_v.at[c], [dsels[k], rowis[g]],
                                       a * b)

            pltpu.async_copy(ods_v.at[c, pl.ds(0, 8)],
                             out_hbm.at[0, cbase + j], sem_w)
            pltpu.async_copy(ods_v.at[c, pl.ds(8, 8)],
                             out_hbm.at[1, cbase + j], sem_w)

            @pl.when(i < NGRP - 1)
            def _():
                pltpu.async_copy(h2_hbm.at[src_v.at[j + DEPTH]],
                                 rs_v.at[c], sem_g)
                pltpu.async_copy(h2_hbm.at[dst_v.at[j + DEPTH]],
                                 rd_v.at[c], sem_g2)
        return 0

    lax.fori_loop(0, NGRP, body, 0)
    for c in range(DEPTH):
        j = (NGRP - 1) * DEPTH + c
        pltpu.make_async_copy(ods_v.at[c, pl.ds(0, 8)],
                              out_hbm.at[0, cbase + j], sem_w).wait()
        pltpu.make_async_copy(ods_v.at[c, pl.ds(8, 8)],
                              out_hbm.at[1, cbase + j], sem_w).wait()


_MB = 1000


def _mm1_body(x_ref, w_ref, deg_ref, h1p_ref, dinv_ref):
    deg = deg_ref[0] + deg_ref[1]
    dinv = lax.rsqrt(deg[:, 0:1] + 1.0)
    h = jnp.dot(x_ref[...], w_ref[...], preferred_element_type=jnp.float32)
    h1p_ref[...] = h * dinv
    dinv_ref[...] = dinv


_mm1 = pl.pallas_call(
    _mm1_body,
    grid=(N // _MB,),
    in_specs=[
        pl.BlockSpec((_MB, D_IN), lambda i: (i, 0)),
        pl.BlockSpec((D_IN, HID), lambda i: (0, 0)),
        pl.BlockSpec((NC, _MB, 16), lambda i: (0, i, 0)),
    ],
    out_specs=[
        pl.BlockSpec((_MB, HID), lambda i: (i, 0)),
        pl.BlockSpec((_MB, 1), lambda i: (i, 0)),
    ],
    out_shape=[
        jax.ShapeDtypeStruct((N, HID), jnp.float32),
        jax.ShapeDtypeStruct((N, 1), jnp.float32),
    ],
)


def _mm2_body(acc_ref, h1p_ref, dinv_ref, w2_ref, b1_ref, out_ref):
    tot = acc_ref[0] + acc_ref[1] + h1p_ref[...]
    h1 = jnp.maximum(tot * dinv_ref[...] + b1_ref[...], 0.0)
    out_ref[...] = jnp.dot(h1, w2_ref[...],
                           preferred_element_type=jnp.float32) * dinv_ref[...]


_mm2 = pl.pallas_call(
    _mm2_body,
    grid=(N // _MB,),
    in_specs=[
        pl.BlockSpec((NC, _MB, HID), lambda i: (0, i, 0)),
        pl.BlockSpec((_MB, HID), lambda i: (i, 0)),
        pl.BlockSpec((_MB, 1), lambda i: (i, 0)),
        pl.BlockSpec((HID, D_EDGE), lambda i: (0, 0)),
        pl.BlockSpec((1, HID), lambda i: (0, 0)),
    ],
    out_specs=pl.BlockSpec((_MB, D_EDGE), lambda i: (i, 0)),
    out_shape=jax.ShapeDtypeStruct((N, D_EDGE), jnp.float32),
)


def _h2_body(acc_ref, h2p_ref, dinv_ref, b2_ref, out_ref):
    tot = acc_ref[0] + acc_ref[1] + h2p_ref[...]
    out_ref[...] = jnp.maximum(tot * dinv_ref[...] + b2_ref[...], 0.0)


_h2fn = pl.pallas_call(
    _h2_body,
    grid=(N // _MB,),
    in_specs=[
        pl.BlockSpec((NC, _MB, D_EDGE), lambda i: (0, i, 0)),
        pl.BlockSpec((_MB, D_EDGE), lambda i: (i, 0)),
        pl.BlockSpec((_MB, 1), lambda i: (i, 0)),
        pl.BlockSpec((1, D_EDGE), lambda i: (0, 0)),
    ],
    out_specs=pl.BlockSpec((_MB, D_EDGE), lambda i: (i, 0)),
    out_shape=jax.ShapeDtypeStruct((N, D_EDGE), jnp.float32),
)

_CB = 64


def _score_body(p_ref, ea_ref, out_ref):
    ea4 = ea_ref[...].reshape(2, 8, _CB, CHUNK).transpose(0, 2, 1, 3)
    q = p_ref[...] * ea4
    out_ref[...] = jax.nn.sigmoid(jnp.sum(q, axis=(0, 2)))


_score = pl.pallas_call(
    _score_body,
    grid=(NCHK // _CB,),
    in_specs=[
        pl.BlockSpec((2, _CB, 8, CHUNK), lambda i: (0, i, 0, 0)),
        pl.BlockSpec((16, _CB * CHUNK), lambda i: (0, i)),
    ],
    out_specs=pl.BlockSpec((_CB, CHUNK), lambda i: (i, 0)),
    out_shape=jax.ShapeDtypeStruct((NCHK, CHUNK), jnp.float32),
)


def kernel(x, edge_index, edge_attr, W1, b1, W2, b2):
    src = edge_index[0]
    dst = edge_index[1]
    pad = E_PAD - E
    srcp = jnp.concatenate(
        [src, jnp.zeros((pad,), jnp.int32)]).reshape(NW, CPT, CHUNK)
    dstp = jnp.concatenate(
        [dst, jnp.full((pad,), N, jnp.int32)]).reshape(NW, CPT, CHUNK)
    dstg = jnp.concatenate(
        [dst, jnp.zeros((pad,), jnp.int32)]).reshape(NW, CPT, CHUNK)

    degp = _deg_kernel(dstp)
    h1p, dinv = _mm1(x, W1, degp)
    acc1 = _agg_kernel(h1p, srcp, dstp)
    h2p = _mm2(acc1, h1p, dinv, W2, b1.reshape(1, HID))
    acc2 = _agg_kernel(h2p, srcp, dstp)
    h2 = _h2fn(acc2, h2p, dinv, b2.reshape(1, D_EDGE))
    pT4 = _gathp_kernel(h2, srcp, dstg)
    score = _score(pT4, edge_attr.T)
    return score.reshape(E_PAD)[:E]


# speedup vs baseline: 1.2636x; 1.2636x over previous
import functools

import jax
import jax.numpy as jnp
from jax import lax
from jax.experimental import pallas as pl
from jax.experimental.pallas import tpu as pltpu
from jax.experimental.pallas import tpu_sc as plsc

N = 10000
E = 320000
D_IN = 256
HID = 16
D_EDGE = 16

NC = 2
NS = 16
NW = NC * NS
CHUNK = 128
CPT = 80
DEPTH = 4
NGRP = CPT // DEPTH
ADEPTH = 8
ANGRP = CPT // ADEPTH
E_PAD = NW * CPT * CHUNK
NCHK = E_PAD // CHUNK
NCHK_R = E // CHUNK
N_PAD = 10112
RPS = N_PAD // NS

_MESH = dict(core_axis_name="c", subcore_axis_name="s")


def _wid():
    return lax.axis_index("s") * NC + lax.axis_index("c")


def _zero_spmem(zbuf_v, acc_sh):
    sid = lax.axis_index("s")

    def fill(i, _):
        zbuf_v[i] = jnp.zeros((16,), jnp.float32)
        return 0

    lax.fori_loop(0, RPS, fill, 0)
    pltpu.sync_copy(zbuf_v, acc_sh.at[pl.ds(sid * RPS, RPS)])


def _write_out(acc_sh, out_hbm):
    cid = lax.axis_index("c")
    sid = lax.axis_index("s")
    pltpu.sync_copy(acc_sh.at[pl.ds(sid * RPS, RPS)],
                    out_hbm.at[cid, pl.ds(sid * RPS, RPS)])


@functools.partial(
    pl.kernel,
    out_type=jax.ShapeDtypeStruct((NC, N_PAD, 16), jnp.float32),
    mesh=plsc.VectorSubcoreMesh(**_MESH),
    compiler_params=pltpu.CompilerParams(use_tc_tiling_on_sc=False),
    scratch_types=[
        pltpu.VMEM((CPT, CHUNK), jnp.int32),
        pltpu.VMEM((CHUNK, 16), jnp.float32),
        pltpu.VMEM((RPS, 16), jnp.float32),
        pltpu.VMEM_SHARED((N_PAD, 16), jnp.float32),
        pltpu.SemaphoreType.DMA,
    ],
)
def _deg_kernel(dst_hbm, out_hbm, dst_v, ones_v, zbuf_v, acc_sh, sem_s):
    wid = _wid()

    def fill(i, _):
        ones_v[i] = jnp.ones((16,), jnp.float32)
        return 0

    lax.fori_loop(0, CHUNK, fill, 0)
    _zero_spmem(zbuf_v, acc_sh)
    plsc.subcore_barrier()
    pltpu.sync_copy(dst_hbm.at[wid], dst_v)

    def body(i, _):
        for c in range(ADEPTH):
            pltpu.async_copy(ones_v, acc_sh.at[dst_v.at[i * ADEPTH + c]],
                             sem_s, add=True)
        for c in range(ADEPTH):
            pltpu.make_async_copy(
                ones_v, acc_sh.at[dst_v.at[i * ADEPTH + c]], sem_s).wait()
        return 0

    lax.fori_loop(0, ANGRP, body, 0)
    plsc.subcore_barrier()
    _write_out(acc_sh, out_hbm)


@functools.partial(
    pl.kernel,
    out_type=jax.ShapeDtypeStruct((NC, N_PAD, 16), jnp.float32),
    mesh=plsc.VectorSubcoreMesh(**_MESH),
    compiler_params=pltpu.CompilerParams(use_tc_tiling_on_sc=False),
    scratch_types=[
        pltpu.VMEM((CPT, CHUNK), jnp.int32),
        pltpu.VMEM((CPT, CHUNK), jnp.int32),
        pltpu.VMEM((ADEPTH, CHUNK, 16), jnp.float32),
        pltpu.VMEM((RPS, 16), jnp.float32),
        pltpu.VMEM_SHARED((N_PAD, 16), jnp.float32),
        pltpu.VMEM_SHARED((N_PAD, 16), jnp.float32),
        pltpu.SemaphoreType.DMA,
        pltpu.SemaphoreType.DMA,
    ],
)
def _agg_kernel(table_hbm, src_hbm, dst_hbm, out_hbm,
                src_v, dst_v, rows_v, zbuf_v, acc_sh, tab_sh, sem_g, sem_s):
    wid = _wid()
    sid = lax.axis_index("s")
    pltpu.sync_copy(table_hbm.at[pl.ds(sid * RPS, RPS)],
                    tab_sh.at[pl.ds(sid * RPS, RPS)])
    _zero_spmem(zbuf_v, acc_sh)
    plsc.subcore_barrier()
    pltpu.sync_copy(src_hbm.at[wid], src_v)
    pltpu.sync_copy(dst_hbm.at[wid], dst_v)

    for c in range(ADEPTH):
        pltpu.async_copy(tab_sh.at[src_v.at[c]], rows_v.at[c], sem_g)

    def body(i, _):
        for c in range(ADEPTH):
            j = i * ADEPTH + c
            pltpu.make_async_copy(
                tab_sh.at[src_v.at[j]], rows_v.at[c], sem_g).wait()
            pltpu.async_copy(rows_v.at[c], acc_sh.at[dst_v.at[j]],
                             sem_s, add=True)
        for c in range(ADEPTH):
            j = i * ADEPTH + c
            pltpu.make_async_copy(
                rows_v.at[c], acc_sh.at[dst_v.at[j]], sem_s).wait()

            @pl.when(i < ANGRP - 1)
            def _():
                pltpu.async_copy(
                    tab_sh.at[src_v.at[j + ADEPTH]], rows_v.at[c], sem_g)
        return 0

    lax.fori_loop(0, ANGRP, body, 0)
    plsc.subcore_barrier()
    _write_out(acc_sh, out_hbm)


@functools.partial(
    pl.kernel,
    out_type=jax.ShapeDtypeStruct((2, NCHK, 8, CHUNK), jnp.float32),
    mesh=plsc.VectorSubcoreMesh(**_MESH),
    compiler_params=pltpu.CompilerParams(use_tc_tiling_on_sc=False,
                                         needs_layout_passes=False),
    scratch_types=[
        pltpu.VMEM((CPT, CHUNK), jnp.int32),
        pltpu.VMEM((CPT, CHUNK), jnp.int32),
        pltpu.VMEM((DEPTH, CHUNK, 16), jnp.float32),
        pltpu.VMEM((DEPTH, CHUNK, 16), jnp.float32),
        pltpu.VMEM((DEPTH, 16, CHUNK), jnp.float32),
        pltpu.VMEM_SHARED((N_PAD, 16), jnp.float32),
        pltpu.SemaphoreType.DMA,
        pltpu.SemaphoreType.DMA,
        pltpu.SemaphoreType.DMA,
    ],
)
def _gathp_kernel(h2_hbm, src_hbm, dst_hbm, out_hbm,
                  src_v, dst_v, rs_v, rd_v, ods_v, tab_sh,
                  sem_g, sem_g2, sem_w):
    wid = _wid()
    sid = lax.axis_index("s")
    pltpu.sync_copy(h2_hbm.at[pl.ds(sid * RPS, RPS)],
                    tab_sh.at[pl.ds(sid * RPS, RPS)])
    pltpu.sync_copy(src_hbm.at[wid], src_v)
    pltpu.sync_copy(dst_hbm.at[wid], dst_v)
    plsc.subcore_barrier()
    cbase = wid * CPT
    iota16 = lax.iota(jnp.int32, 16)

    for c in range(DEPTH):
        pltpu.async_copy(tab_sh.at[src_v.at[c]], rs_v.at[c], sem_g)
        pltpu.async_copy(tab_sh.at[dst_v.at[c]], rd_v.at[c], sem_g2)

    def body(i, _):
        for c in range(DEPTH):
            j = i * DEPTH + c

            @pl.when(i > 0)
            def _():
                pltpu.make_async_copy(
                    ods_v.at[c, pl.ds(0, 8)],
                    out_hbm.at[0, cbase + j - DEPTH], sem_w).wait()
                pltpu.make_async_copy(
                    ods_v.at[c, pl.ds(8, 8)],
                    out_hbm.at[1, cbase + j - DEPTH], sem_w).wait()

            pltpu.make_async_copy(
                tab_sh.at[src_v.at[j]], rs_v.at[c], sem_g).wait()
            pltpu.make_async_copy(
                tab_sh.at[dst_v.at[j]], rd_v.at[c], sem_g2).wait()

            def gloop(g, _):
                rowi = g * 16 + iota16
                for dd in range(16):
                    cold = jnp.full((16,), dd, dtype=jnp.int32)
                    a = plsc.load_gather(rs_v.at[c], [rowi, cold])
                    b = plsc.load_gather(rd_v.at[c], [rowi, cold])
                    ods_v[c, dd, pl.ds(g * 16, 16)] = a * b
                return 0

            lax.fori_loop(0, 8, gloop, 0)

            pltpu.async_copy(ods_v.at[c, pl.ds(0, 8)],
                             out_hbm.at[0, cbase + j], sem_w)
            pltpu.async_copy(ods_v.at[c, pl.ds(8, 8)],
                             out_hbm.at[1, cbase + j], sem_w)

            @pl.when(i < NGRP - 1)
            def _():
                pltpu.async_copy(tab_sh.at[src_v.at[j + DEPTH]],
                                 rs_v.at[c], sem_g)
                pltpu.async_copy(tab_sh.at[dst_v.at[j + DEPTH]],
                                 rd_v.at[c], sem_g2)
        return 0

    lax.fori_loop(0, NGRP, body, 0)
    for c in range(DEPTH):
        j = (NGRP - 1) * DEPTH + c
        pltpu.make_async_copy(ods_v.at[c, pl.ds(0, 8)],
                              out_hbm.at[0, cbase + j], sem_w).wait()
        pltpu.make_async_copy(ods_v.at[c, pl.ds(8, 8)],
                              out_hbm.at[1, cbase + j], sem_w).wait()


_MB = RPS


def _mm1_body(x_ref, w_ref, deg_ref, h1p_ref, dinv_ref):
    deg = deg_ref[0] + deg_ref[1]
    dinv = lax.rsqrt(deg[:, 0:1] + 1.0)
    h = jnp.dot(x_ref[...], w_ref[...], preferred_element_type=jnp.float32)
    h1p_ref[...] = h * dinv
    dinv_ref[...] = dinv


_mm1 = pl.pallas_call(
    _mm1_body,
    grid=(N_PAD // _MB,),
    in_specs=[
        pl.BlockSpec((_MB, D_IN), lambda i: (i, 0)),
        pl.BlockSpec((D_IN, HID), lambda i: (0, 0)),
        pl.BlockSpec((NC, _MB, 16), lambda i: (0, i, 0)),
    ],
    out_specs=[
        pl.BlockSpec((_MB, HID), lambda i: (i, 0)),
        pl.BlockSpec((_MB, 1), lambda i: (i, 0)),
    ],
    out_shape=[
        jax.ShapeDtypeStruct((N_PAD, HID), jnp.float32),
        jax.ShapeDtypeStruct((N_PAD, 1), jnp.float32),
    ],
)


def _mm2_body(acc_ref, h1p_ref, dinv_ref, w2_ref, b1_ref, out_ref):
    tot = acc_ref[0] + acc_ref[1] + h1p_ref[...]
    h1 = jnp.maximum(tot * dinv_ref[...] + b1_ref[...], 0.0)
    out_ref[...] = jnp.dot(h1, w2_ref[...],
                           preferred_element_type=jnp.float32) * dinv_ref[...]


_mm2 = pl.pallas_call(
    _mm2_body,
    grid=(N_PAD // _MB,),
    in_specs=[
        pl.BlockSpec((NC, _MB, HID), lambda i: (0, i, 0)),
        pl.BlockSpec((_MB, HID), lambda i: (i, 0)),
        pl.BlockSpec((_MB, 1), lambda i: (i, 0)),
        pl.BlockSpec((HID, D_EDGE), lambda i: (0, 0)),
        pl.BlockSpec((1, HID), lambda i: (0, 0)),
    ],
    out_specs=pl.BlockSpec((_MB, D_EDGE), lambda i: (i, 0)),
    out_shape=jax.ShapeDtypeStruct((N_PAD, D_EDGE), jnp.float32),
)


def _h2_body(acc_ref, h2p_ref, dinv_ref, b2_ref, out_ref):
    tot = acc_ref[0] + acc_ref[1] + h2p_ref[...]
    out_ref[...] = jnp.maximum(tot * dinv_ref[...] + b2_ref[...], 0.0)


_h2fn = pl.pallas_call(
    _h2_body,
    grid=(N_PAD // _MB,),
    in_specs=[
        pl.BlockSpec((NC, _MB, D_EDGE), lambda i: (0, i, 0)),
        pl.BlockSpec((_MB, D_EDGE), lambda i: (i, 0)),
        pl.BlockSpec((_MB, 1), lambda i: (i, 0)),
        pl.BlockSpec((1, D_EDGE), lambda i: (0, 0)),
    ],
    out_specs=pl.BlockSpec((_MB, D_EDGE), lambda i: (i, 0)),
    out_shape=jax.ShapeDtypeStruct((N_PAD, D_EDGE), jnp.float32),
)

_CB = 64


def _score_body(p_ref, ea_ref, out_ref):
    ea4 = ea_ref[...].reshape(2, 8, _CB, CHUNK).transpose(0, 2, 1, 3)
    q = p_ref[...] * ea4
    out_ref[...] = jax.nn.sigmoid(jnp.sum(q, axis=(0, 2)))


_score = pl.pallas_call(
    _score_body,
    grid=(NCHK // _CB,),
    in_specs=[
        pl.BlockSpec((2, _CB, 8, CHUNK), lambda i: (0, i, 0, 0)),
        pl.BlockSpec((16, _CB * CHUNK), lambda i: (0, i)),
    ],
    out_specs=pl.BlockSpec((_CB, CHUNK), lambda i: (i, 0)),
    out_shape=jax.ShapeDtypeStruct((NCHK, CHUNK), jnp.float32),
)


def kernel(x, edge_index, edge_attr, W1, b1, W2, b2):
    src = edge_index[0]
    dst = edge_index[1]
    pad = E_PAD - E
    srcp = jnp.concatenate(
        [src, jnp.zeros((pad,), jnp.int32)]).reshape(NW, CPT, CHUNK)
    dstp = jnp.concatenate(
        [dst, jnp.full((pad,), N, jnp.int32)]).reshape(NW, CPT, CHUNK)
    dstg = jnp.concatenate(
        [dst, jnp.zeros((pad,), jnp.int32)]).reshape(NW, CPT, CHUNK)

    degp = _deg_kernel(dstp)
    h1p, dinv = _mm1(x, W1, degp)
    acc1 = _agg_kernel(h1p, srcp, dstp)
    h2p = _mm2(acc1, h1p, dinv, W2, b1.reshape(1, HID))
    acc2 = _agg_kernel(h2p, srcp, dstp)
    h2 = _h2fn(acc2, h2p, dinv, b2.reshape(1, D_EDGE))
    p1 = _gathp_kernel(h2, srcp, dstg)
    score = _score(p1, edge_attr.T)
    return score.reshape(E_PAD)[:E]


# speedup vs baseline: 1.5890x; 1.2575x over previous
import functools

import jax
import jax.numpy as jnp
from jax import lax
from jax.experimental import pallas as pl
from jax.experimental.pallas import tpu as pltpu
from jax.experimental.pallas import tpu_sc as plsc

N = 10000
E = 320000
D_IN = 256
HID = 16
D_EDGE = 16

NC = 2
NS = 16
NW = NC * NS
CHUNK = 128
CPT = 80
DEPTH = 4
NGRP = CPT // DEPTH
ADEPTH = 8
ANGRP = CPT // ADEPTH
E_PAD = NW * CPT * CHUNK
NCHK = E_PAD // CHUNK
NCHK_R = E // CHUNK
N_PAD = 10112
RPS = N_PAD // NS

_MESH = dict(core_axis_name="c", subcore_axis_name="s")


def _wid():
    return lax.axis_index("s") * NC + lax.axis_index("c")


def _zero_spmem(zbuf_v, acc_sh):
    sid = lax.axis_index("s")

    def fill(i, _):
        zbuf_v[i] = jnp.zeros((16,), jnp.float32)
        return 0

    lax.fori_loop(0, RPS, fill, 0)
    pltpu.sync_copy(zbuf_v, acc_sh.at[pl.ds(sid * RPS, RPS)])


def _write_out(acc_sh, out_hbm):
    cid = lax.axis_index("c")
    sid = lax.axis_index("s")
    pltpu.sync_copy(acc_sh.at[pl.ds(sid * RPS, RPS)],
                    out_hbm.at[cid, pl.ds(sid * RPS, RPS)])


@functools.partial(
    pl.kernel,
    out_type=jax.ShapeDtypeStruct((NC, N_PAD, 16), jnp.float32),
    mesh=plsc.VectorSubcoreMesh(**_MESH),
    compiler_params=pltpu.CompilerParams(use_tc_tiling_on_sc=False),
    scratch_types=[
        pltpu.VMEM((CPT, CHUNK), jnp.int32),
        pltpu.VMEM((CHUNK, 16), jnp.float32),
        pltpu.VMEM((RPS, 16), jnp.float32),
        pltpu.VMEM_SHARED((N_PAD, 16), jnp.float32),
        pltpu.SemaphoreType.DMA,
    ],
)
def _deg_kernel(dst_hbm, out_hbm, dst_v, ones_v, zbuf_v, acc_sh, sem_s):
    wid = _wid()

    def fill(i, _):
        ones_v[i] = jnp.ones((16,), jnp.float32)
        return 0

    lax.fori_loop(0, CHUNK, fill, 0)
    _zero_spmem(zbuf_v, acc_sh)
    plsc.subcore_barrier()
    pltpu.sync_copy(dst_hbm.at[wid], dst_v)

    def body(i, _):
        for c in range(ADEPTH):
            pltpu.async_copy(ones_v, acc_sh.at[dst_v.at[i * ADEPTH + c]],
                             sem_s, add=True)
        for c in range(ADEPTH):
            pltpu.make_async_copy(
                ones_v, acc_sh.at[dst_v.at[i * ADEPTH + c]], sem_s).wait()
        return 0

    lax.fori_loop(0, ANGRP, body, 0)
    plsc.subcore_barrier()
    _write_out(acc_sh, out_hbm)


@functools.partial(
    pl.kernel,
    out_type=jax.ShapeDtypeStruct((NC, N_PAD, 16), jnp.float32),
    mesh=plsc.VectorSubcoreMesh(**_MESH),
    compiler_params=pltpu.CompilerParams(use_tc_tiling_on_sc=False),
    scratch_types=[
        pltpu.VMEM((CPT, CHUNK), jnp.int32),
        pltpu.VMEM((CPT, CHUNK), jnp.int32),
        pltpu.VMEM((ADEPTH, CHUNK, 16), jnp.float32),
        pltpu.VMEM((RPS, 16), jnp.float32),
        pltpu.VMEM_SHARED((N_PAD, 16), jnp.float32),
        pltpu.VMEM_SHARED((N_PAD, 16), jnp.float32),
        pltpu.SemaphoreType.DMA,
        pltpu.SemaphoreType.DMA,
    ],
)
def _agg_kernel(table_hbm, src_hbm, dst_hbm, out_hbm,
                src_v, dst_v, rows_v, zbuf_v, acc_sh, tab_sh, sem_g, sem_s):
    wid = _wid()
    sid = lax.axis_index("s")
    pltpu.sync_copy(table_hbm.at[pl.ds(sid * RPS, RPS)],
                    tab_sh.at[pl.ds(sid * RPS, RPS)])
    _zero_spmem(zbuf_v, acc_sh)
    plsc.subcore_barrier()
    pltpu.sync_copy(src_hbm.at[wid], src_v)
    pltpu.sync_copy(dst_hbm.at[wid], dst_v)

    for c in range(ADEPTH):
        pltpu.async_copy(tab_sh.at[src_v.at[c]], rows_v.at[c], sem_g)

    def body(i, _):
        for c in range(ADEPTH):
            j = i * ADEPTH + c
            pltpu.make_async_copy(
                tab_sh.at[src_v.at[j]], rows_v.at[c], sem_g).wait()
            pltpu.async_copy(rows_v.at[c], acc_sh.at[dst_v.at[j]],
                             sem_s, add=True)
        for c in range(ADEPTH):
            j = i * ADEPTH + c
            pltpu.make_async_copy(
                rows_v.at[c], acc_sh.at[dst_v.at[j]], sem_s).wait()

            @pl.when(i < ANGRP - 1)
            def _():
                pltpu.async_copy(
                    tab_sh.at[src_v.at[j + ADEPTH]], rows_v.at[c], sem_g)
        return 0

    lax.fori_loop(0, ANGRP, body, 0)
    plsc.subcore_barrier()
    _write_out(acc_sh, out_hbm)


@functools.partial(
    pl.kernel,
    out_type=jax.ShapeDtypeStruct((2, NCHK, 8, CHUNK), jnp.float32),
    mesh=plsc.VectorSubcoreMesh(**_MESH),
    compiler_params=pltpu.CompilerParams(use_tc_tiling_on_sc=False,
                                         needs_layout_passes=False),
    scratch_types=[
        pltpu.VMEM((CPT, CHUNK), jnp.int32),
        pltpu.VMEM((CPT, CHUNK), jnp.int32),
        pltpu.VMEM((DEPTH, CHUNK, 16), jnp.float32),
        pltpu.VMEM((DEPTH, CHUNK, 16), jnp.float32),
        pltpu.VMEM((DEPTH, 16, CHUNK), jnp.float32),
        pltpu.VMEM_SHARED((N_PAD, 16), jnp.float32),
        pltpu.SemaphoreType.DMA,
        pltpu.SemaphoreType.DMA,
        pltpu.SemaphoreType.DMA,
    ],
)
def _gathp_kernel(h2_hbm, src_hbm, dst_hbm, out_hbm,
                  src_v, dst_v, rs_v, rd_v, ods_v, tab_sh,
                  sem_g, sem_g2, sem_w):
    wid = _wid()
    sid = lax.axis_index("s")
    pltpu.sync_copy(h2_hbm.at[pl.ds(sid * RPS, RPS)],
                    tab_sh.at[pl.ds(sid * RPS, RPS)])
    pltpu.sync_copy(src_hbm.at[wid], src_v)
    pltpu.sync_copy(dst_hbm.at[wid], dst_v)
    plsc.subcore_barrier()
    cbase = wid * CPT
    iota16 = lax.iota(jnp.int32, 16)
    dsels = [(iota16 + k) & 15 for k in range(16)]

    for c in range(DEPTH):
        pltpu.async_copy(tab_sh.at[src_v.at[c]], rs_v.at[c], sem_g)
        pltpu.async_copy(tab_sh.at[dst_v.at[c]], rd_v.at[c], sem_g2)

    def body(i, _):
        for c in range(DEPTH):
            j = i * DEPTH + c

            @pl.when(i > 0)
            def _():
                pltpu.make_async_copy(
                    ods_v.at[c, pl.ds(0, 8)],
                    out_hbm.at[0, cbase + j - DEPTH], sem_w).wait()
                pltpu.make_async_copy(
                    ods_v.at[c, pl.ds(8, 8)],
                    out_hbm.at[1, cbase + j - DEPTH], sem_w).wait()

            pltpu.make_async_copy(
                tab_sh.at[src_v.at[j]], rs_v.at[c], sem_g).wait()
            pltpu.make_async_copy(
                tab_sh.at[dst_v.at[j]], rd_v.at[c], sem_g2).wait()

            def gloop(g, _):
                rowi = g * 16 + iota16
                for k in range(16):
                    a = plsc.load_gather(rs_v.at[c], [rowi, dsels[k]])
                    b = plsc.load_gather(rd_v.at[c], [rowi, dsels[k]])
                    plsc.store_scatter(ods_v.at[c], [dsels[k], rowi], a * b)
                return 0

            lax.fori_loop(0, 8, gloop, 0)

            pltpu.async_copy(ods_v.at[c, pl.ds(0, 8)],
                             out_hbm.at[0, cbase + j], sem_w)
            pltpu.async_copy(ods_v.at[c, pl.ds(8, 8)],
                             out_hbm.at[1, cbase + j], sem_w)

            @pl.when(i < NGRP - 1)
            def _():
                pltpu.async_copy(tab_sh.at[src_v.at[j + DEPTH]],
                                 rs_v.at[c], sem_g)
                pltpu.async_copy(tab_sh.at[dst_v.at[j + DEPTH]],
                                 rd_v.at[c], sem_g2)
        return 0

    lax.fori_loop(0, NGRP, body, 0)
    for c in range(DEPTH):
        j = (NGRP - 1) * DEPTH + c
        pltpu.make_async_copy(ods_v.at[c, pl.ds(0, 8)],
                              out_hbm.at[0, cbase + j], sem_w).wait()
        pltpu.make_async_copy(ods_v.at[c, pl.ds(8, 8)],
                              out_hbm.at[1, cbase + j], sem_w).wait()


_MB = RPS


def _mm1_body(x_ref, w_ref, deg_ref, h1p_ref, dinv_ref):
    deg = deg_ref[0] + deg_ref[1]
    dinv = lax.rsqrt(deg[:, 0:1] + 1.0)
    h = jnp.dot(x_ref[...], w_ref[...], preferred_element_type=jnp.float32)
    h1p_ref[...] = h * dinv
    dinv_ref[...] = dinv


_mm1 = pl.pallas_call(
    _mm1_body,
    grid=(N_PAD // _MB,),
    in_specs=[
        pl.BlockSpec((_MB, D_IN), lambda i: (i, 0)),
        pl.BlockSpec((D_IN, HID), lambda i: (0, 0)),
        pl.BlockSpec((NC, _MB, 16), lambda i: (0, i, 0)),
    ],
    out_specs=[
        pl.BlockSpec((_MB, HID), lambda i: (i, 0)),
        pl.BlockSpec((_MB, 1), lambda i: (i, 0)),
    ],
    out_shape=[
        jax.ShapeDtypeStruct((N_PAD, HID), jnp.float32),
        jax.ShapeDtypeStruct((N_PAD, 1), jnp.float32),
    ],
)


def _mm2_body(acc_ref, h1p_ref, dinv_ref, w2_ref, b1_ref, out_ref):
    tot = acc_ref[0] + acc_ref[1] + h1p_ref[...]
    h1 = jnp.maximum(tot * dinv_ref[...] + b1_ref[...], 0.0)
    out_ref[...] = jnp.dot(h1, w2_ref[...],
                           preferred_element_type=jnp.float32) * dinv_ref[...]


_mm2 = pl.pallas_call(
    _mm2_body,
    grid=(N_PAD // _MB,),
    in_specs=[
        pl.BlockSpec((NC, _MB, HID), lambda i: (0, i, 0)),
        pl.BlockSpec((_MB, HID), lambda i: (i, 0)),
        pl.BlockSpec((_MB, 1), lambda i: (i, 0)),
        pl.BlockSpec((HID, D_EDGE), lambda i: (0, 0)),
        pl.BlockSpec((1, HID), lambda i: (0, 0)),
    ],
    out_specs=pl.BlockSpec((_MB, D_EDGE), lambda i: (i, 0)),
    out_shape=jax.ShapeDtypeStruct((N_PAD, D_EDGE), jnp.float32),
)


def _h2_body(acc_ref, h2p_ref, dinv_ref, b2_ref, out_ref):
    tot = acc_ref[0] + acc_ref[1] + h2p_ref[...]
    out_ref[...] = jnp.maximum(tot * dinv_ref[...] + b2_ref[...], 0.0)


_h2fn = pl.pallas_call(
    _h2_body,
    grid=(N_PAD // _MB,),
    in_specs=[
        pl.BlockSpec((NC, _MB, D_EDGE), lambda i: (0, i, 0)),
        pl.BlockSpec((_MB, D_EDGE), lambda i: (i, 0)),
        pl.BlockSpec((_MB, 1), lambda i: (i, 0)),
        pl.BlockSpec((1, D_EDGE), lambda i: (0, 0)),
    ],
    out_specs=pl.BlockSpec((_MB, D_EDGE), lambda i: (i, 0)),
    out_shape=jax.ShapeDtypeStruct((N_PAD, D_EDGE), jnp.float32),
)

_CB = 64


def _score_body(p_ref, ea_ref, out_ref):
    ea4 = ea_ref[...].reshape(2, 8, _CB, CHUNK).transpose(0, 2, 1, 3)
    q = p_ref[...] * ea4
    out_ref[...] = jax.nn.sigmoid(jnp.sum(q, axis=(0, 2)))


_score = pl.pallas_call(
    _score_body,
    grid=(NCHK // _CB,),
    in_specs=[
        pl.BlockSpec((2, _CB, 8, CHUNK), lambda i: (0, i, 0, 0)),
        pl.BlockSpec((16, _CB * CHUNK), lambda i: (0, i)),
    ],
    out_specs=pl.BlockSpec((_CB, CHUNK), lambda i: (i, 0)),
    out_shape=jax.ShapeDtypeStruct((NCHK, CHUNK), jnp.float32),
)


def kernel(x, edge_index, edge_attr, W1, b1, W2, b2):
    src = edge_index[0]
    dst = edge_index[1]
    pad = E_PAD - E
    srcp = jnp.concatenate(
        [src, jnp.zeros((pad,), jnp.int32)]).reshape(NW, CPT, CHUNK)
    dstp = jnp.concatenate(
        [dst, jnp.full((pad,), N, jnp.int32)]).reshape(NW, CPT, CHUNK)
    dstg = jnp.concatenate(
        [dst, jnp.zeros((pad,), jnp.int32)]).reshape(NW, CPT, CHUNK)

    degp = _deg_kernel(dstp)
    h1p, dinv = _mm1(x, W1, degp)
    acc1 = _agg_kernel(h1p, srcp, dstp)
    h2p = _mm2(acc1, h1p, dinv, W2, b1.reshape(1, HID))
    acc2 = _agg_kernel(h2p, srcp, dstp)
    h2 = _h2fn(acc2, h2p, dinv, b2.reshape(1, D_EDGE))
    p1 = _gathp_kernel(h2, srcp, dstg)
    score = _score(p1, edge_attr.T)
    return score.reshape(E_PAD)[:E]
